# Initial kernel scaffold; baseline (speedup 1.0000x reference)
import jax
import jax.numpy as jnp
from jax.experimental import pallas as pl


def kernel(x, edge_index, W1_l, b1, W1_r, W2_l, b2, W2_r):
    raise NotImplementedError("write your pallas kernel here")


# speedup vs baseline: 7.6779x; 7.6779x over previous
import functools

import jax
import jax.numpy as jnp
from jax import lax
from jax.experimental import pallas as pl
from jax.experimental.pallas import tpu as pltpu
from jax.experimental.pallas import tpu_sc as plsc

N = 10000
D = 128
E = 320000
NC = 2
NS = 16
NW = NC * NS
EPT = E // NW
K = 80
NCHUNK = 128
HC = 64
NP = N + 8
STRIPE = 624
SCH = 48
REM = N - NS * STRIPE
BR = 1000


def _dotT(a, w):
    return lax.dot_general(a, w, (((1,), (1,)), ((), ())),
                           preferred_element_type=jnp.float32)


def _tc_pre(x, Wl, Wr, b):
    def body(x_ref, wl_ref, wr_ref, b_ref, p_ref, r_ref):
        xb = x_ref[...]
        p_ref[...] = _dotT(xb, wl_ref[...])
        r_ref[...] = _dotT(xb, wr_ref[...]) + b_ref[...]

    return pl.pallas_call(
        body,
        grid=(N // BR,),
        in_specs=[
            pl.BlockSpec((BR, D), lambda i: (i, 0)),
            pl.BlockSpec((D, D), lambda i: (0, 0)),
            pl.BlockSpec((D, D), lambda i: (0, 0)),
            pl.BlockSpec((1, D), lambda i: (0, 0)),
        ],
        out_specs=[pl.BlockSpec((BR, D), lambda i: (i, 0))] * 2,
        out_shape=[jax.ShapeDtypeStruct((N, D), jnp.float32)] * 2,
    )(x, Wl, Wr, b.reshape(1, D))


def _tc_mid(s1a, s1b, ca, cb, r1, Wl, Wr, b2):
    def body(sa, sb, ca_r, cb_r, r1_r, wl, wr, b_r, p2, r2):
        cnt = ca_r[...][:, :1] + cb_r[...][:, :1]
        inv = 1.0 / jnp.maximum(cnt, 1.0)
        h = jnp.maximum((sa[...] + sb[...]) * inv + r1_r[...], 0.0)
        p2[...] = _dotT(h, wl[...])
        r2[...] = _dotT(h, wr[...]) + b_r[...]

    return pl.pallas_call(
        body,
        grid=(N // BR,),
        in_specs=[
            pl.BlockSpec((BR, D), lambda i: (i, 0)),
            pl.BlockSpec((BR, D), lambda i: (i, 0)),
            pl.BlockSpec((BR, D), lambda i: (i, 0)),
            pl.BlockSpec((BR, D), lambda i: (i, 0)),
            pl.BlockSpec((BR, D), lambda i: (i, 0)),
            pl.BlockSpec((D, D), lambda i: (0, 0)),
            pl.BlockSpec((D, D), lambda i: (0, 0)),
            pl.BlockSpec((1, D), lambda i: (0, 0)),
        ],
        out_specs=[pl.BlockSpec((BR, D), lambda i: (i, 0))] * 2,
        out_shape=[jax.ShapeDtypeStruct((N, D), jnp.float32)] * 2,
    )(s1a, s1b, ca, cb, r1, Wl, Wr, b2.reshape(1, D))


def _tc_post(s2a, s2b, ca, cb, r2):
    def body(sa, sb, ca_r, cb_r, r2_r, out):
        cnt = ca_r[...][:, :1] + cb_r[...][:, :1]
        inv = 1.0 / jnp.maximum(cnt, 1.0)
        o = (sa[...] + sb[...]) * inv + r2_r[...]
        m = jnp.max(o, axis=1, keepdims=True)
        e = jnp.exp(o - m)
        lse = jnp.log(jnp.sum(e, axis=1, keepdims=True))
        out[...] = o - m - lse

    return pl.pallas_call(
        body,
        grid=(N // BR,),
        in_specs=[
            pl.BlockSpec((BR, D), lambda i: (i, 0)),
            pl.BlockSpec((BR, D), lambda i: (i, 0)),
            pl.BlockSpec((BR, D), lambda i: (i, 0)),
            pl.BlockSpec((BR, D), lambda i: (i, 0)),
            pl.BlockSpec((BR, D), lambda i: (i, 0)),
        ],
        out_specs=pl.BlockSpec((BR, D), lambda i: (i, 0)),
        out_shape=jax.ShapeDtypeStruct((N, D), jnp.float32),
    )(s2a, s2b, ca, cb, r2)


def _sc_agg(P, src3, dst3):
    mesh = plsc.VectorSubcoreMesh(core_axis_name="c", subcore_axis_name="s", num_cores=NC, num_subcores=NS)

    out_type = jax.ShapeDtypeStruct((NC, N, D), jnp.float32)

    scratch = [
        pltpu.VMEM_SHARED((NP, D), jnp.float32),
        pltpu.VMEM((HC, K), jnp.int32),
        pltpu.VMEM((HC, K), jnp.int32),
        pltpu.VMEM((K, D), jnp.float32),
        pltpu.VMEM((K, D), jnp.float32),
        pltpu.VMEM((SCH, D), jnp.float32),
        pltpu.SemaphoreType.DMA,
        pltpu.SemaphoreType.DMA,
    ]

    def body(P_hbm, src_hbm, dst_hbm, out_hbm, acc, src_v, dst_v, rows_a,
             rows_b, zbuf, sem_a, sem_b):
        c = lax.axis_index("c")
        s = lax.axis_index("s")
        wid = s * NC + c
        row0 = s * STRIPE

        zero16 = jnp.zeros((16,), jnp.float32)

        def zrow(i, carry):
            for j in range(D // 16):
                zbuf[i, pl.ds(j * 16, 16)] = zero16
            return carry

        lax.fori_loop(0, SCH, zrow, 0)

        for t in range(STRIPE // SCH):
            pltpu.sync_copy(zbuf, acc.at[pl.ds(row0 + t * SCH, SCH)])

        @pl.when(s == 0)
        def _zero_rem():
            pltpu.sync_copy(zbuf.at[pl.ds(0, REM)], acc.at[pl.ds(NS * STRIPE, REM)])

        plsc.subcore_barrier()

        for h in range(NCHUNK // HC):
            pltpu.sync_copy(src_hbm.at[wid, pl.ds(h * HC, HC)], src_v)
            pltpu.sync_copy(dst_hbm.at[wid, pl.ds(h * HC, HC)], dst_v)

            def pair(i, carry):
                j = 2 * i
                da = pltpu.async_copy(P_hbm.at[src_v.at[j]], rows_a, sem_a)
                db = pltpu.async_copy(P_hbm.at[src_v.at[j + 1]], rows_b, sem_b)
                da.wait()
                pltpu.sync_copy(rows_a, acc.at[dst_v.at[j]], add=True)
                db.wait()
                pltpu.sync_copy(rows_b, acc.at[dst_v.at[j + 1]], add=True)
                return carry

            lax.fori_loop(0, HC // 2, pair, 0)

        plsc.subcore_barrier()

        for t in range(STRIPE // SCH):
            r = row0 + t * SCH
            pltpu.sync_copy(acc.at[pl.ds(r, SCH)], zbuf)
            pltpu.sync_copy(zbuf, out_hbm.at[c, pl.ds(r, SCH)])

        @pl.when(s == 0)
        def _wb_rem():
            pltpu.sync_copy(acc.at[pl.ds(NS * STRIPE, REM)], zbuf.at[pl.ds(0, REM)])
            pltpu.sync_copy(zbuf.at[pl.ds(0, REM)], out_hbm.at[c, pl.ds(NS * STRIPE, REM)])

    run = pl.kernel(body, out_type=out_type, mesh=mesh, scratch_types=scratch)
    return run(P, src3, dst3)


def _sc_cnt(dst3):
    mesh = plsc.VectorSubcoreMesh(core_axis_name="c", subcore_axis_name="s", num_cores=NC, num_subcores=NS)

    out_type = jax.ShapeDtypeStruct((NC, N, D), jnp.float32)

    scratch = [
        pltpu.VMEM_SHARED((NP, D), jnp.float32),
        pltpu.VMEM((NCHUNK, K), jnp.int32),
        pltpu.VMEM((K, D), jnp.float32),
        pltpu.VMEM((SCH, D), jnp.float32),
    ]

    def body(dst_hbm, cnt_hbm, cntacc, dst_v, ones_v, zbuf):
        c = lax.axis_index("c")
        s = lax.axis_index("s")
        wid = s * NC + c
        row0 = s * STRIPE

        pltpu.sync_copy(dst_hbm.at[wid], dst_v)

        zero16 = jnp.zeros((16,), jnp.float32)
        one16 = jnp.ones((16,), jnp.float32)

        def orow(i, carry):
            for j in range(D // 16):
                ones_v[i, pl.ds(j * 16, 16)] = one16
            return carry

        lax.fori_loop(0, K, orow, 0)

        def zrow(i, carry):
            for j in range(D // 16):
                zbuf[i, pl.ds(j * 16, 16)] = zero16
            return carry

        lax.fori_loop(0, SCH, zrow, 0)

        for t in range(STRIPE // SCH):
            pltpu.sync_copy(zbuf, cntacc.at[pl.ds(row0 + t * SCH, SCH)])

        @pl.when(s == 0)
        def _zero_rem():
            pltpu.sync_copy(zbuf.at[pl.ds(0, REM)], cntacc.at[pl.ds(NS * STRIPE, REM)])

        plsc.subcore_barrier()

        def chunk(j, carry):
            pltpu.sync_copy(ones_v, cntacc.at[dst_v.at[j]], add=True)
            return carry

        lax.fori_loop(0, NCHUNK, chunk, 0)
        plsc.subcore_barrier()

        for t in range(STRIPE // SCH):
            r = row0 + t * SCH
            pltpu.sync_copy(cntacc.at[pl.ds(r, SCH)], zbuf)
            pltpu.sync_copy(zbuf, cnt_hbm.at[c, pl.ds(r, SCH)])

        @pl.when(s == 0)
        def _wb_rem():
            pltpu.sync_copy(cntacc.at[pl.ds(NS * STRIPE, REM)], zbuf.at[pl.ds(0, REM)])
            pltpu.sync_copy(zbuf.at[pl.ds(0, REM)], cnt_hbm.at[c, pl.ds(NS * STRIPE, REM)])

    run = pl.kernel(body, out_type=out_type, mesh=mesh, scratch_types=scratch)
    return run(dst3)


def kernel(x, edge_index, W1_l, b1, W1_r, W2_l, b2, W2_r):
    pad = NCHUNK * K - EPT
    src_fill = jnp.broadcast_to(jnp.arange(pad, dtype=jnp.int32) % N, (NW, pad))
    src2 = jnp.concatenate([edge_index[0].reshape(NW, EPT), src_fill], axis=1)
    dst2 = jnp.pad(edge_index[1].reshape(NW, EPT), ((0, 0), (0, pad)),
                   constant_values=N)
    src3 = src2.reshape(NW, NCHUNK, K)
    dst3 = dst2.reshape(NW, NCHUNK, K)

    C = _sc_cnt(dst3)
    P1, R1 = _tc_pre(x, W1_l, W1_r, b1)
    S1 = _sc_agg(P1, src3, dst3)
    P2, R2 = _tc_mid(S1[0], S1[1], C[0], C[1], R1, W2_l, W2_r, b2)
    S2 = _sc_agg(P2, src3, dst3)
    return _tc_post(S2[0], S2[1], C[0], C[1], R2)
